# Initial kernel scaffold; baseline (speedup 1.0000x reference)
import jax
import jax.numpy as jnp
from jax.experimental import pallas as pl


def kernel(x, kmer_pos, word_table, pos_table, kmer_w):
    raise NotImplementedError("write your pallas kernel here")


# speedup vs baseline: 1.2446x; 1.2446x over previous
import functools
import math

import jax
import jax.numpy as jnp
from jax import lax
from jax.experimental import pallas as pl
from jax.experimental.pallas import tpu as pltpu
from jax.experimental.pallas import tpu_sc as plsc

NC = 2
NS = 16
NW = NC * NS
L = 16
G = 128


@functools.cache
def _build(B, S, V, D):
    rows = B * S
    rpw = rows // NW
    ng = rpw // G
    wpb = NW // B
    inv = 1.0 / math.sqrt(D)
    nj = D // L

    mesh = plsc.VectorSubcoreMesh(core_axis_name="c", subcore_axis_name="s")

    @functools.partial(
        pl.kernel,
        mesh=mesh,
        out_type=jax.ShapeDtypeStruct((rows, D), jnp.float32),
        scratch_types=[
            pltpu.VMEM((ng, G), jnp.int32),
            pltpu.VMEM((rpw, D), jnp.float32),
            pltpu.VMEM((rpw, D), jnp.float32),
            pltpu.VMEM((L,), jnp.float32),
            pltpu.VMEM((D,), jnp.float32),
            pltpu.SemaphoreType.DMA,
        ],
    )
    def sc_kernel(x_hbm, kpos_hbm, table_hbm, pos_hbm, kw_hbm, out_hbm,
                  idx_v, rows_v, pos_v, kp_v, kw_v, sem):
        wid = lax.axis_index("s") * NC + lax.axis_index("c")
        base = wid * rpw
        s0 = (wid % wpb) * rpw

        pltpu.sync_copy(x_hbm.at[wid], idx_v)
        copies = [
            pltpu.async_copy(table_hbm.at[idx_v.at[g]],
                             rows_v.at[pl.ds(g * G, G), :], sem)
            for g in range(ng)
        ]
        pltpu.sync_copy(pos_hbm.at[pl.ds(s0, rpw), :], pos_v)
        pltpu.sync_copy(kpos_hbm.at[wid], kp_v)
        pltpu.sync_copy(kw_hbm, kw_v)

        kpb = kp_v[...]
        bias = [(kpb * kw_v[pl.ds(j * L, L)]) * inv for j in range(nj)]

        for cp in copies:
            cp.wait()

        def body(r, carry):
            for j in range(nj):
                sl = pl.ds(j * L, L)
                rows_v[r, sl] = (rows_v[r, sl]
                                 + pos_v[r, sl] * inv
                                 + bias[j])
            return carry

        lax.fori_loop(0, rpw, body, 0)

        pltpu.sync_copy(rows_v, out_hbm.at[pl.ds(base, rpw), :])

    return sc_kernel


@jax.jit
def kernel(x, kmer_pos, word_table, pos_table, kmer_w):
    B, S = x.shape
    V, D = word_table.shape
    rows = B * S
    rpw = rows // NW
    ng = rpw // G

    x_idx = x.reshape(NW, ng, G).astype(jnp.int32)
    wpb = NW // B
    kp_rep = jnp.broadcast_to(
        jnp.repeat(kmer_pos[:, 0].astype(jnp.float32), wpb)[:, None], (NW, L))
    kw_flat = kmer_w[:, 0].astype(jnp.float32)

    out = _build(B, S, V, D)(x_idx, kp_rep, word_table, pos_table, kw_flat)
    return out.reshape(B, S, D)


# speedup vs baseline: 1.3146x; 1.0562x over previous
import functools
import math

import jax
import jax.numpy as jnp
from jax import lax
from jax.experimental import pallas as pl
from jax.experimental.pallas import tpu as pltpu
from jax.experimental.pallas import tpu_sc as plsc

NC = 2
NS = 16
NW = NC * NS
L = 16
C = 64


@functools.cache
def _build(B, S, V, D):
    rows = B * S
    rpw = rows // NW
    nch = rpw // C
    wpb = NW // B
    inv = 1.0 / math.sqrt(D)
    nj = D // L

    mesh = plsc.VectorSubcoreMesh(core_axis_name="c", subcore_axis_name="s")

    @functools.partial(
        pl.kernel,
        mesh=mesh,
        out_type=jax.ShapeDtypeStruct((rows, D), jnp.float32),
        scratch_types=[
            pltpu.VMEM((nch, C), jnp.int32),
            pltpu.VMEM((rpw, D), jnp.float32),
            pltpu.VMEM((rpw, D), jnp.float32),
            pltpu.VMEM((L + D,), jnp.float32),
            *([pltpu.SemaphoreType.DMA] * nch),
            *([pltpu.SemaphoreType.DMA] * nch),
            pltpu.SemaphoreType.DMA,
        ],
    )
    def sc_kernel(x_hbm, table_hbm, pos_hbm, kbuf_hbm, out_hbm,
                  idx_v, rows_v, pos_v, kbuf_v, *sems):
        gsem = sems[:nch]
        psem = sems[nch:2 * nch]
        osem = sems[2 * nch]

        wid = lax.axis_index("s") * NC + lax.axis_index("c")
        base = wid * rpw
        s0 = (wid % wpb) * rpw

        pltpu.sync_copy(x_hbm.at[wid], idx_v)
        gcp = []
        pcp = []
        for c in range(nch):
            sl = pl.ds(c * C, C)
            gcp.append(pltpu.async_copy(table_hbm.at[idx_v.at[c]],
                                        rows_v.at[sl, :], gsem[c]))
            pcp.append(pltpu.async_copy(pos_hbm.at[pl.ds(s0 + c * C, C), :],
                                        pos_v.at[sl, :], psem[c]))

        pltpu.sync_copy(kbuf_hbm.at[wid], kbuf_v)
        kpb = kbuf_v[pl.ds(0, L)]
        bias = [(kpb * kbuf_v[pl.ds(L + j * L, L)]) * inv for j in range(nj)]

        ocp = []
        for c in range(nch):
            gcp[c].wait()
            pcp[c].wait()

            def body(r, carry):
                for j in range(nj):
                    sl = pl.ds(j * L, L)
                    rows_v[r, sl] = (rows_v[r, sl]
                                     + pos_v[r, sl] * inv
                                     + bias[j])
                return carry

            lax.fori_loop(c * C, (c + 1) * C, body, 0)
            ocp.append(pltpu.async_copy(
                rows_v.at[pl.ds(c * C, C), :],
                out_hbm.at[pl.ds(base + c * C, C), :], osem))
        for cp in ocp:
            cp.wait()

    return sc_kernel


@jax.jit
def kernel(x, kmer_pos, word_table, pos_table, kmer_w):
    B, S = x.shape
    V, D = word_table.shape
    rows = B * S
    rpw = rows // NW
    nch = rpw // C

    x_idx = x.reshape(NW, nch, C).astype(jnp.int32)
    wpb = NW // B
    kp_rep = jnp.broadcast_to(
        jnp.repeat(kmer_pos[:, 0].astype(jnp.float32), wpb)[:, None], (NW, L))
    kw_rep = jnp.broadcast_to(kmer_w[:, 0].astype(jnp.float32)[None, :],
                              (NW, D))
    kbuf = jnp.concatenate([kp_rep, kw_rep], axis=1)

    out = _build(B, S, V, D)(x_idx, word_table, pos_table, kbuf)
    return out.reshape(B, S, D)
